# baseline (speedup 1.0000x reference)
import functools

import jax
import jax.numpy as jnp
from jax import lax
from jax.experimental import pallas as pl
from jax.experimental.pallas import tpu as pltpu
from jax.experimental.pallas import tpu_sc as plsc

BATCH = 4096
FP_DIM = 2048
LATENT_DIM = 128

_info = plsc.get_sparse_core_info()
_NC, _NS = _info.num_cores, _info.num_subcores
_NW = _NC * _NS
_B_PER_W = BATCH // _NW

_BB = 1024
_CB = 2048


@functools.partial(
    pl.kernel,
    mesh=plsc.VectorSubcoreMesh(core_axis_name="c", subcore_axis_name="s"),
    out_type=jax.ShapeDtypeStruct((BATCH, LATENT_DIM), jnp.float32),
    scratch_types=[
        pltpu.VMEM((_B_PER_W,), jnp.int32),
        pltpu.VMEM((_B_PER_W, LATENT_DIM), jnp.float32),
        pltpu.SemaphoreType.DMA,
    ],
)
def _sc_gather(idx_hbm, table_hbm, out_hbm, idx_v, rows_v, sem):
    wid = lax.axis_index("s") * _NC + lax.axis_index("c")
    base = wid * _B_PER_W
    pltpu.sync_copy(idx_hbm.at[pl.ds(base, _B_PER_W)], idx_v)
    pltpu.async_copy(table_hbm.at[idx_v], rows_v, sem).wait()
    pltpu.sync_copy(rows_v, out_hbm.at[pl.ds(base, _B_PER_W)])


def _matmul_body(ds_ref, smiles_ref, w_ref, b_ref, p_ref):
    drug = jnp.dot(smiles_ref[...], w_ref[...], preferred_element_type=jnp.float32)
    p_ref[...] = ds_ref[0] * (drug + b_ref[...])


def _tc_matmul(ds, smiles, w, b2d):
    return pl.pallas_call(
        _matmul_body,
        grid=(BATCH // _BB,),
        in_specs=[
            pl.BlockSpec(memory_space=pltpu.SMEM),
            pl.BlockSpec((_BB, FP_DIM), lambda i: (i, 0)),
            pl.BlockSpec((FP_DIM, LATENT_DIM), lambda i: (0, 0)),
            pl.BlockSpec((1, LATENT_DIM), lambda i: (0, 0)),
        ],
        out_specs=pl.BlockSpec((_BB, LATENT_DIM), lambda i: (i, 0)),
        out_shape=jax.ShapeDtypeStruct((BATCH, LATENT_DIM), jnp.float32),
        compiler_params=pltpu.CompilerParams(
            dimension_semantics=("parallel",),
        ),
    )(ds, smiles, w, b2d)


def _combine_body(cs_ref, emb_ref, p_ref, o_ref):
    o_ref[...] = cs_ref[0] * emb_ref[...] + p_ref[...]


def _tc_combine(cs, cell_emb, p):
    return pl.pallas_call(
        _combine_body,
        grid=(BATCH // _CB,),
        in_specs=[
            pl.BlockSpec(memory_space=pltpu.SMEM),
            pl.BlockSpec((_CB, LATENT_DIM), lambda i: (i, 0)),
            pl.BlockSpec((_CB, LATENT_DIM), lambda i: (i, 0)),
        ],
        out_specs=pl.BlockSpec((_CB, LATENT_DIM), lambda i: (i, 0)),
        out_shape=jax.ShapeDtypeStruct((BATCH, LATENT_DIM), jnp.float32),
        compiler_params=pltpu.CompilerParams(
            dimension_semantics=("parallel",),
        ),
    )(cs, cell_emb, p)


def kernel(cell_type, smiles, cell_table, W_mol, b_mol, cell_scale, drug_scale):
    idx = cell_type.astype(jnp.int32)
    cell_emb = _sc_gather(idx, cell_table)
    p = _tc_matmul(drug_scale.reshape(1), smiles, W_mol, b_mol.reshape(1, LATENT_DIM))
    return _tc_combine(cell_scale.reshape(1), cell_emb, p)
